# baseline (speedup 1.0000x reference)
import functools

import jax
import jax.numpy as jnp
from jax import lax
from jax.experimental import pallas as pl
from jax.experimental.pallas import tpu as pltpu
from jax.experimental.pallas import tpu_sc as plsc

_EPS = 1e-12


@functools.lru_cache(maxsize=None)
def _make_sc_gather(N: int, V: int, D: int):
    info = plsc.get_sparse_core_info()
    NC, NS = info.num_cores, info.num_subcores
    NW = NC * NS
    assert N % NW == 0
    b_per_w = N // NW
    CH = 8
    for c in range(8, min(512, b_per_w) + 1, 8):
        if b_per_w % c == 0:
            CH = c
    n_ch = b_per_w // CH
    mesh = plsc.VectorSubcoreMesh(core_axis_name="c", subcore_axis_name="s")

    @functools.partial(
        pl.kernel,
        mesh=mesh,
        out_type=jax.ShapeDtypeStruct((N * D // 2,), jnp.int32),
        scratch_types=[
            pltpu.VMEM((CH,), jnp.int32),
            pltpu.VMEM((CH, D), jnp.float32),
            pltpu.VMEM((CH * D // 2,), jnp.int32),
            pltpu.SemaphoreType.DMA,
        ],
    )
    def gather_kernel(idx_hbm, table_hbm, out_hbm, idx_v, rows_v, rows_pk,
                      sem):
        wid = lax.axis_index("s") * NC + lax.axis_index("c")
        base = wid * b_per_w

        def body(i, carry):
            off = base + i * CH
            pltpu.sync_copy(idx_hbm.at[pl.ds(off, CH)], idx_v)
            pltpu.async_copy(table_hbm.at[idx_v], rows_v, sem).wait()

            def conv_row(j, c2):
                for q in range(D // 32):
                    a = rows_v[j, pl.ds(q * 32, 16)]
                    b = rows_v[j, pl.ds(q * 32 + 16, 16)]
                    ai = lax.bitcast_convert_type(a, jnp.int32)
                    bi = lax.bitcast_convert_type(b, jnp.int32)
                    lo = lax.shift_right_logical(ai + 0x8000, 16)
                    hi = lax.bitwise_and(bi + 0x8000, jnp.int32(-65536))
                    rows_pk[pl.ds(j * (D // 2) + q * 16, 16)] = (
                        lax.bitwise_or(lo, hi))
                return c2

            lax.fori_loop(0, CH, conv_row, 0)
            pltpu.sync_copy(rows_pk,
                            out_hbm.at[pl.ds(off * (D // 2), CH * (D // 2))])
            return carry

        lax.fori_loop(0, n_ch, body, 0)

    return gather_kernel


def _proj_ln_body(g_ref, w_ref, gamma_ref, beta_ref, _full_ref, o_ref):
    T2 = g_ref.shape[0]
    gi = g_ref[...]
    lo = lax.bitcast_convert_type(lax.shift_left(gi, 16), jnp.float32)
    hi = lax.bitcast_convert_type(
        lax.bitwise_and(gi, jnp.int32(-65536)), jnp.float32)
    w = w_ref[...]
    gam = gamma_ref[...]
    bet = beta_ref[...]
    for half in (0, 1):
        cs = half * 64
        x = jnp.concatenate([lo[:, cs:cs + 64], hi[:, cs:cs + 64]], axis=1)
        h = jnp.dot(x, w, preferred_element_type=jnp.float32)
        mean = jnp.mean(h, axis=-1, keepdims=True)
        c = h - mean
        var = jnp.mean(c * c, axis=-1, keepdims=True)
        o_ref[pl.ds(half * T2, T2), :] = (
            (c * lax.rsqrt(var + _EPS)) * gam + bet)


def _proj_ln_slice(g_s, W, gamma, beta, out_prev, base_blk, T, alias, N, H):
    N2, D2 = g_s.shape
    return pl.pallas_call(
        _proj_ln_body,
        grid=(2 * N2 // T,),
        in_specs=[
            pl.BlockSpec((T // 2, D2), lambda i: (i, 0)),
            pl.BlockSpec((D2, H), lambda i: (0, 0)),
            pl.BlockSpec((1, H), lambda i: (0, 0)),
            pl.BlockSpec((1, H), lambda i: (0, 0)),
            pl.BlockSpec(memory_space=pl.ANY),
        ],
        out_specs=pl.BlockSpec((T, H), lambda i: (i + base_blk, 0)),
        out_shape=jax.ShapeDtypeStruct((N, H), jnp.float32),
        input_output_aliases={4: 0} if alias else {},
    )(g_s, W, gamma.reshape(1, H), beta.reshape(1, H), out_prev)


def kernel(entity_ids, table, W, gamma, beta):
    B, L = entity_ids.shape
    N = B * L
    V, D = table.shape
    H = W.shape[1]
    T = 4096
    idx_lm = entity_ids.T.reshape(N).astype(jnp.int32)
    m = jnp.arange(N, dtype=jnp.int32)
    j = m % T
    f = (m - j) + (j % 2) * (T // 2) + j // 2
    idx = idx_lm[f]
    perm = ([32 * (c // 16) + (c % 16) for c in range(64)]
            + [32 * (c // 16) + 16 + (c % 16) for c in range(64)])
    Wp = W[jnp.array(perm, dtype=jnp.int32), :]
    slice_blks = [8, 42]
    offs, acc = [], 0
    for nb in slice_blks:
        offs.append(acc)
        acc += nb * T
    g_slices = [
        _make_sc_gather(nb * T, V, D)(
            lax.slice_in_dim(idx, off, off + nb * T), table).reshape(
                nb * T // 2, D)
        for nb, off in zip(slice_blks, offs)
    ]
    out = g_slices[0]
    for s, nb in enumerate(slice_blks):
        out = _proj_ln_slice(g_slices[s], Wp, gamma, beta, out,
                             base_blk=offs[s] // T, T=T, alias=s > 0,
                             N=N, H=H)
    return out.reshape(L, B, H).transpose(1, 0, 2)


# speedup vs baseline: 1.3529x; 1.3529x over previous
import functools

import jax
import jax.numpy as jnp
from jax import lax
from jax.experimental import pallas as pl
from jax.experimental.pallas import tpu as pltpu
from jax.experimental.pallas import tpu_sc as plsc

_EPS = 1e-12


@functools.lru_cache(maxsize=None)
def _make_sc_gather(N: int, V: int, D: int):
    info = plsc.get_sparse_core_info()
    NC, NS = info.num_cores, info.num_subcores
    NW = NC * NS
    assert N % NW == 0
    b_per_w = N // NW
    CH = 8
    for c in range(8, min(512, b_per_w) + 1, 8):
        if b_per_w % c == 0:
            CH = c
    n_ch = b_per_w // CH
    mesh = plsc.VectorSubcoreMesh(core_axis_name="c", subcore_axis_name="s")

    @functools.partial(
        pl.kernel,
        mesh=mesh,
        out_type=jax.ShapeDtypeStruct((N, D), jnp.float32),
        scratch_types=[
            pltpu.VMEM((CH,), jnp.int32),
            pltpu.VMEM((CH, D), jnp.float32),
            pltpu.SemaphoreType.DMA,
        ],
    )
    def gather_kernel(idx_hbm, table_hbm, out_hbm, idx_v, rows_v, sem):
        wid = lax.axis_index("s") * NC + lax.axis_index("c")
        base = wid * b_per_w

        def body(i, carry):
            off = base + i * CH
            pltpu.sync_copy(idx_hbm.at[pl.ds(off, CH)], idx_v)
            pltpu.async_copy(table_hbm.at[idx_v], rows_v, sem).wait()
            pltpu.sync_copy(rows_v, out_hbm.at[pl.ds(off, CH)])
            return carry

        lax.fori_loop(0, n_ch, body, 0)

    return gather_kernel


def _proj_ln_body(g_ref, w_ref, gamma_ref, beta_ref, _full_ref, o_ref):
    g = g_ref[...]
    w = w_ref[...]
    h = jnp.dot(g, w, preferred_element_type=jnp.float32)
    mean = jnp.mean(h, axis=-1, keepdims=True)
    c = h - mean
    var = jnp.mean(c * c, axis=-1, keepdims=True)
    o_ref[...] = (c * lax.rsqrt(var + _EPS)) * gamma_ref[...] + beta_ref[...]


def _proj_ln_slice(g_s, W, gamma, beta, out_prev, base_blk, T, alias, N, H):
    NS_, D = g_s.shape
    return pl.pallas_call(
        _proj_ln_body,
        grid=(NS_ // T,),
        in_specs=[
            pl.BlockSpec((T, D), lambda i: (i, 0)),
            pl.BlockSpec((D, H), lambda i: (0, 0)),
            pl.BlockSpec((1, H), lambda i: (0, 0)),
            pl.BlockSpec((1, H), lambda i: (0, 0)),
            pl.BlockSpec(memory_space=pl.ANY),
        ],
        out_specs=pl.BlockSpec((T, H), lambda i: (i + base_blk, 0)),
        out_shape=jax.ShapeDtypeStruct((N, H), jnp.float32),
        input_output_aliases={4: 0} if alias else {},
    )(g_s, W, gamma.reshape(1, H), beta.reshape(1, H), out_prev)


def kernel(entity_ids, table, W, gamma, beta):
    B, L = entity_ids.shape
    N = B * L
    V, D = table.shape
    H = W.shape[1]
    T = 4096
    idx = entity_ids.T.reshape(N).astype(jnp.int32)
    slice_blks = [8, 42]
    offs, acc = [], 0
    for nb in slice_blks:
        offs.append(acc)
        acc += nb * T
    g_slices = [
        _make_sc_gather(nb * T, V, D)(
            lax.slice_in_dim(idx, off, off + nb * T), table)
        for nb, off in zip(slice_blks, offs)
    ]
    out = g_slices[0]
    for s, nb in enumerate(slice_blks):
        out = _proj_ln_slice(g_slices[s], W, gamma, beta, out,
                             base_blk=offs[s] // T, T=T, alias=s > 0,
                             N=N, H=H)
    return out.reshape(L, B, H).transpose(1, 0, 2)
